# Initial kernel scaffold; baseline (speedup 1.0000x reference)
import jax
import jax.numpy as jnp
from jax.experimental import pallas as pl


def kernel(h, edge_index, edge_type, edge_attr, W1, b1, W2, b2, gru_w_ih, gru_w_hh, gru_b_ih, gru_b_hh):
    raise NotImplementedError("write your pallas kernel here")


# speedup vs baseline: 1.8133x; 1.8133x over previous
import functools

import jax
import jax.numpy as jnp
from jax import lax
from jax.experimental import pallas as pl
from jax.experimental.pallas import tpu as pltpu
from jax.experimental.pallas import tpu_sc as plsc

_NC = 2
_NS = 16
_NW = _NC * _NS


def _sc_gather(table, idx):
    n, d = table.shape
    e = idx.shape[0]
    per_w = e // _NW
    ch = 80
    n_ch = per_w // ch
    assert per_w % ch == 0 and per_w * _NW == e

    mesh = plsc.VectorSubcoreMesh(core_axis_name="c", subcore_axis_name="s",
                                  num_cores=_NC, num_subcores=_NS)

    @functools.partial(
        pl.kernel,
        out_type=jax.ShapeDtypeStruct((e, d), jnp.int32),
        mesh=mesh,
        scratch_types=[
            pltpu.VMEM((ch,), jnp.int32),
            pltpu.VMEM((ch, d), jnp.int32),
            pltpu.SemaphoreType.DMA,
        ],
        compiler_params=pltpu.CompilerParams(use_tc_tiling_on_sc=False),
    )
    def k(table_hbm, idx_hbm, out_hbm, idx_v, rows_v, sem):
        wid = lax.axis_index("s") * _NC + lax.axis_index("c")
        base = wid * per_w

        def body(c, carry):
            off = base + c * ch
            pltpu.sync_copy(idx_hbm.at[pl.ds(off, ch)], idx_v)
            pltpu.async_copy(table_hbm.at[idx_v], rows_v, sem).wait()
            pltpu.sync_copy(rows_v, out_hbm.at[pl.ds(off, ch)])
            return carry

        lax.fori_loop(0, n_ch, body, 0)

    return k(table, idx)


def _sc_scatter_add(msgs, dst, zeros, n_pad):
    e, d = msgs.shape
    per_w = e // _NW
    ch = 80
    n_ch = per_w // ch
    rows_per_tile = n_pad // _NS
    assert per_w % ch == 0 and rows_per_tile % 8 == 0

    mesh = plsc.VectorSubcoreMesh(core_axis_name="c", subcore_axis_name="s",
                                  num_cores=_NC, num_subcores=_NS)

    @functools.partial(
        pl.kernel,
        out_type=jax.ShapeDtypeStruct((_NC, n_pad, d), jnp.float32),
        mesh=mesh,
        scratch_types=[
            pltpu.VMEM((ch,), jnp.int32),
            pltpu.VMEM((ch, d), jnp.float32),
            pltpu.VMEM_SHARED((n_pad, d), jnp.float32),
        ],
    )
    def k(msgs_hbm, dst_hbm, zeros_hbm, out_hbm, idx_v, rows_v, acc_sh):
        cid = lax.axis_index("c")
        sid = lax.axis_index("s")
        wid = sid * _NC + cid
        my_rows = pl.ds(sid * rows_per_tile, rows_per_tile)

        pltpu.sync_copy(zeros_hbm, acc_sh.at[my_rows])
        plsc.subcore_barrier()

        base = wid * per_w

        def body(c, carry):
            off = base + c * ch
            pltpu.sync_copy(dst_hbm.at[pl.ds(off, ch)], idx_v)
            pltpu.sync_copy(msgs_hbm.at[pl.ds(off, ch)], rows_v)
            pltpu.sync_copy(rows_v, acc_sh.at[idx_v], add=True)
            return carry

        lax.fori_loop(0, n_ch, body, 0)
        plsc.subcore_barrier()
        pltpu.sync_copy(acc_sh.at[my_rows], out_hbm.at[cid, my_rows])

    return k(msgs, dst, zeros)


def _tc_mlp(ty_f32, h_src, edge_attr, w1h, w1a, b1, w2, b2):
    e, h = h_src.shape
    t = w1h.shape[0]
    ea = edge_attr.shape[1]
    be = 512
    grid = (e // be,)

    def body(ty_ref, hs_ref, ea_ref, w1h_ref, w1a_ref, b1_ref, w2_ref, b2_ref,
             out_ref):
        hs = hs_ref[...]
        eat = ea_ref[...]
        ty = ty_ref[...]
        acc = jnp.zeros((be, h), jnp.float32)
        for tt in range(t):
            x1 = jnp.dot(hs, w1h_ref[tt], preferred_element_type=jnp.float32)
            x1 = x1 + jnp.dot(eat, w1a_ref[tt],
                              preferred_element_type=jnp.float32)
            x1 = x1 + b1_ref[tt][None, :]
            y = jnp.maximum(x1, 0.0).astype(jnp.bfloat16)
            m = jnp.dot(y, w2_ref[tt], preferred_element_type=jnp.float32)
            m = m + b2_ref[tt][None, :]
            acc = acc + jnp.where(ty == float(tt), m, 0.0)
        out_ref[...] = acc

    return pl.pallas_call(
        body,
        grid=grid,
        in_specs=[
            pl.BlockSpec((be, 1), lambda i: (i, 0)),
            pl.BlockSpec((be, h), lambda i: (i, 0)),
            pl.BlockSpec((be, ea), lambda i: (i, 0)),
            pl.BlockSpec((t, h, h), lambda i: (0, 0, 0)),
            pl.BlockSpec((t, ea, h), lambda i: (0, 0, 0)),
            pl.BlockSpec((t, h), lambda i: (0, 0)),
            pl.BlockSpec((t, h, h), lambda i: (0, 0, 0)),
            pl.BlockSpec((t, h), lambda i: (0, 0)),
        ],
        out_specs=pl.BlockSpec((be, h), lambda i: (i, 0)),
        out_shape=jax.ShapeDtypeStruct((e, h), jnp.float32),
    )(ty_f32, h_src, edge_attr, w1h, w1a, b1, w2, b2)


def _tc_gru(msg_a, msg_b, h, w_ih, w_hh, b_ih, b_hh):
    n, hd = h.shape
    bn = 1000
    grid = (n // bn,)

    def body(ma_ref, mb_ref, h_ref, wih_ref, whh_ref, bih_ref, bhh_ref,
             out_ref):
        msg = ma_ref[...] + mb_ref[...]
        hv = h_ref[...]
        dims = (((1,), (1,)), ((), ()))
        gi = lax.dot_general(msg.astype(jnp.bfloat16), wih_ref[...], dims,
                             preferred_element_type=jnp.float32)
        gi = gi + bih_ref[...]
        gh = lax.dot_general(hv.astype(jnp.bfloat16), whh_ref[...], dims,
                             preferred_element_type=jnp.float32)
        gh = gh + bhh_ref[...]
        r = jax.nn.sigmoid(gi[:, :hd] + gh[:, :hd])
        z = jax.nn.sigmoid(gi[:, hd:2 * hd] + gh[:, hd:2 * hd])
        nn = jnp.tanh(gi[:, 2 * hd:] + r * gh[:, 2 * hd:])
        out_ref[...] = (1.0 - z) * nn + z * hv

    return pl.pallas_call(
        body,
        grid=grid,
        in_specs=[
            pl.BlockSpec((bn, hd), lambda i: (i, 0)),
            pl.BlockSpec((bn, hd), lambda i: (i, 0)),
            pl.BlockSpec((bn, hd), lambda i: (i, 0)),
            pl.BlockSpec((3 * hd, hd), lambda i: (0, 0)),
            pl.BlockSpec((3 * hd, hd), lambda i: (0, 0)),
            pl.BlockSpec((1, 3 * hd), lambda i: (0, 0)),
            pl.BlockSpec((1, 3 * hd), lambda i: (0, 0)),
        ],
        out_specs=pl.BlockSpec((bn, hd), lambda i: (i, 0)),
        out_shape=jax.ShapeDtypeStruct((n, hd), jnp.float32),
    )(msg_a, msg_b, h, w_ih, w_hh, b_ih, b_hh)


def kernel(h, edge_index, edge_type, edge_attr, W1, b1, W2, b2,
           gru_w_ih, gru_w_hh, gru_b_ih, gru_b_hh):
    n, hd = h.shape
    e = edge_index.shape[1]

    src = edge_index[0].astype(jnp.int32)
    dst = edge_index[1].astype(jnp.int32)
    ty_f32 = edge_type.astype(jnp.float32).reshape(e, 1)

    h_bf = h.astype(jnp.bfloat16)
    h_i32 = lax.bitcast_convert_type(h_bf.reshape(n, hd // 2, 2), jnp.int32)
    hsrc_i32 = _sc_gather(h_i32, src)
    h_src = lax.bitcast_convert_type(hsrc_i32, jnp.bfloat16).reshape(e, hd)

    w1h = W1[:, :hd, :].astype(jnp.bfloat16)
    w1a = W1[:, hd:, :].astype(jnp.bfloat16)
    w2 = W2.astype(jnp.bfloat16)
    msgs = _tc_mlp(ty_f32, h_src, edge_attr.astype(jnp.bfloat16),
                   w1h, w1a, b1, w2, b2)

    n_pad = ((n + 8 * _NS - 1) // (8 * _NS)) * (8 * _NS)
    zeros = jnp.zeros((n_pad // _NS, hd), jnp.float32)
    partials = _sc_scatter_add(msgs, dst, zeros, n_pad)

    return _tc_gru(partials[0, :n], partials[1, :n], h,
                   gru_w_ih.astype(jnp.bfloat16), gru_w_hh.astype(jnp.bfloat16),
                   gru_b_ih.reshape(1, 3 * hd), gru_b_hh.reshape(1, 3 * hd))


# speedup vs baseline: 4.0562x; 2.2369x over previous
import functools

import jax
import jax.numpy as jnp
from jax import lax
from jax.experimental import pallas as pl
from jax.experimental.pallas import tpu as pltpu
from jax.experimental.pallas import tpu_sc as plsc

_NC = 2
_NS = 16
_NW = _NC * _NS


def _sc_gather(table, idx):
    n, d = table.shape
    e = idx.shape[0]
    per_w = e // _NW
    ch = 80
    n_ch = per_w // ch
    assert per_w % ch == 0 and per_w * _NW == e

    mesh = plsc.VectorSubcoreMesh(core_axis_name="c", subcore_axis_name="s",
                                  num_cores=_NC, num_subcores=_NS)

    @functools.partial(
        pl.kernel,
        out_type=jax.ShapeDtypeStruct((e, d), jnp.float32),
        mesh=mesh,
        scratch_types=[
            pltpu.VMEM((ch,), jnp.int32),
            pltpu.VMEM((ch, d), jnp.float32),
            pltpu.SemaphoreType.DMA,
        ],
    )
    def k(table_hbm, idx_hbm, out_hbm, idx_v, rows_v, sem):
        wid = lax.axis_index("s") * _NC + lax.axis_index("c")
        base = wid * per_w

        def body(c, carry):
            off = base + c * ch
            pltpu.sync_copy(idx_hbm.at[pl.ds(off, ch)], idx_v)
            pltpu.async_copy(table_hbm.at[idx_v], rows_v, sem).wait()
            pltpu.sync_copy(rows_v, out_hbm.at[pl.ds(off, ch)])
            return carry

        lax.fori_loop(0, n_ch, body, 0)

    return k(table, idx)


def _sc_scatter_add(msgs, dst, zeros, n_pad):
    e, d = msgs.shape
    per_w = e // _NW
    ch = 80
    n_ch = per_w // ch
    rows_per_tile = n_pad // _NS
    assert per_w % ch == 0 and rows_per_tile % 8 == 0

    mesh = plsc.VectorSubcoreMesh(core_axis_name="c", subcore_axis_name="s",
                                  num_cores=_NC, num_subcores=_NS)

    @functools.partial(
        pl.kernel,
        out_type=jax.ShapeDtypeStruct((_NC, n_pad, d), jnp.float32),
        mesh=mesh,
        scratch_types=[
            pltpu.VMEM((ch,), jnp.int32),
            pltpu.VMEM((ch, d), jnp.float32),
            pltpu.VMEM_SHARED((n_pad, d), jnp.float32),
        ],
    )
    def k(msgs_hbm, dst_hbm, zeros_hbm, out_hbm, idx_v, rows_v, acc_sh):
        cid = lax.axis_index("c")
        sid = lax.axis_index("s")
        wid = sid * _NC + cid
        my_rows = pl.ds(sid * rows_per_tile, rows_per_tile)

        pltpu.sync_copy(zeros_hbm, acc_sh.at[my_rows])
        plsc.subcore_barrier()

        base = wid * per_w

        def body(c, carry):
            off = base + c * ch
            pltpu.sync_copy(dst_hbm.at[pl.ds(off, ch)], idx_v)
            pltpu.sync_copy(msgs_hbm.at[pl.ds(off, ch)], rows_v)
            pltpu.sync_copy(rows_v, acc_sh.at[idx_v], add=True)
            return carry

        lax.fori_loop(0, n_ch, body, 0)
        plsc.subcore_barrier()
        pltpu.sync_copy(acc_sh.at[my_rows], out_hbm.at[cid, my_rows])

    return k(msgs, dst, zeros)


def _tc_mlp(ty_f32, h_src, edge_attr, w1h_cat, w1a_cat, b1_cat, w2_cat, b2):
    e, h = h_src.shape
    t = b2.shape[0]
    ea = edge_attr.shape[1]
    th = t * h
    be = 1280
    assert e % be == 0
    grid = (e // be,)

    def body(ty_ref, hs_ref, ea_ref, w1h_ref, w1a_ref, b1_ref, w2_ref, b2_ref,
             out_ref):
        hs = hs_ref[...].astype(jnp.bfloat16)
        eat = ea_ref[...].astype(jnp.bfloat16)
        ty = ty_ref[...]
        x1 = jnp.dot(hs, w1h_ref[...], preferred_element_type=jnp.float32)
        x1 = x1 + jnp.dot(eat, w1a_ref[...],
                          preferred_element_type=jnp.float32)
        x1 = x1 + b1_ref[...]
        x1 = jnp.maximum(x1, 0.0)
        colt = (lax.broadcasted_iota(jnp.int32, (1, th), 1) // h)
        xm = jnp.where(colt.astype(jnp.float32) == ty, x1, 0.0)
        m = jnp.dot(xm.astype(jnp.bfloat16), w2_ref[...],
                    preferred_element_type=jnp.float32)
        tyrow = lax.broadcasted_iota(jnp.int32, (1, t), 1).astype(jnp.float32)
        m4 = jnp.where(tyrow == ty, 1.0, 0.0)
        m = m + jnp.dot(m4.astype(jnp.bfloat16),
                        b2_ref[...].astype(jnp.bfloat16),
                        preferred_element_type=jnp.float32)
        out_ref[...] = m

    return pl.pallas_call(
        body,
        grid=grid,
        in_specs=[
            pl.BlockSpec((be, 1), lambda i: (i, 0)),
            pl.BlockSpec((be, h), lambda i: (i, 0)),
            pl.BlockSpec((be, ea), lambda i: (i, 0)),
            pl.BlockSpec((h, th), lambda i: (0, 0)),
            pl.BlockSpec((ea, th), lambda i: (0, 0)),
            pl.BlockSpec((1, th), lambda i: (0, 0)),
            pl.BlockSpec((th, h), lambda i: (0, 0)),
            pl.BlockSpec((t, h), lambda i: (0, 0)),
        ],
        out_specs=pl.BlockSpec((be, h), lambda i: (i, 0)),
        out_shape=jax.ShapeDtypeStruct((e, h), jnp.float32),
    )(ty_f32, h_src, edge_attr, w1h_cat, w1a_cat, b1_cat, w2_cat, b2)


def _tc_gru(msg_a, msg_b, h, w_ih, w_hh, b_ih, b_hh):
    n, hd = h.shape
    bn = 1000
    grid = (n // bn,)

    def body(ma_ref, mb_ref, h_ref, wih_ref, whh_ref, bih_ref, bhh_ref,
             out_ref):
        msg = ma_ref[...] + mb_ref[...]
        hv = h_ref[...]
        dims = (((1,), (1,)), ((), ()))
        gi = lax.dot_general(msg.astype(jnp.bfloat16), wih_ref[...], dims,
                             preferred_element_type=jnp.float32)
        gi = gi + bih_ref[...]
        gh = lax.dot_general(hv.astype(jnp.bfloat16), whh_ref[...], dims,
                             preferred_element_type=jnp.float32)
        gh = gh + bhh_ref[...]
        r = jax.nn.sigmoid(gi[:, :hd] + gh[:, :hd])
        z = jax.nn.sigmoid(gi[:, hd:2 * hd] + gh[:, hd:2 * hd])
        nn = jnp.tanh(gi[:, 2 * hd:] + r * gh[:, 2 * hd:])
        out_ref[...] = (1.0 - z) * nn + z * hv

    return pl.pallas_call(
        body,
        grid=grid,
        in_specs=[
            pl.BlockSpec((bn, hd), lambda i: (i, 0)),
            pl.BlockSpec((bn, hd), lambda i: (i, 0)),
            pl.BlockSpec((bn, hd), lambda i: (i, 0)),
            pl.BlockSpec((3 * hd, hd), lambda i: (0, 0)),
            pl.BlockSpec((3 * hd, hd), lambda i: (0, 0)),
            pl.BlockSpec((1, 3 * hd), lambda i: (0, 0)),
            pl.BlockSpec((1, 3 * hd), lambda i: (0, 0)),
        ],
        out_specs=pl.BlockSpec((bn, hd), lambda i: (i, 0)),
        out_shape=jax.ShapeDtypeStruct((n, hd), jnp.float32),
    )(msg_a, msg_b, h, w_ih, w_hh, b_ih, b_hh)


def kernel(h, edge_index, edge_type, edge_attr, W1, b1, W2, b2,
           gru_w_ih, gru_w_hh, gru_b_ih, gru_b_hh):
    n, hd = h.shape
    e = edge_index.shape[1]

    src = edge_index[0].astype(jnp.int32)
    dst = edge_index[1].astype(jnp.int32)
    ty_f32 = edge_type.astype(jnp.float32).reshape(e, 1)

    h_src = _sc_gather(h, src)

    t = W1.shape[0]
    th = t * hd
    w1h_cat = W1[:, :hd, :].transpose(1, 0, 2).reshape(hd, th)
    w1a_cat = W1[:, hd:, :].transpose(1, 0, 2).reshape(-1, th)
    b1_cat = b1.reshape(1, th)
    w2_cat = W2.reshape(th, hd)
    msgs = _tc_mlp(ty_f32, h_src, edge_attr,
                   w1h_cat.astype(jnp.bfloat16), w1a_cat.astype(jnp.bfloat16),
                   b1_cat, w2_cat.astype(jnp.bfloat16), b2)

    n_pad = ((n + 8 * _NS - 1) // (8 * _NS)) * (8 * _NS)
    zeros = jnp.zeros((n_pad // _NS, hd), jnp.float32)
    partials = _sc_scatter_add(msgs, dst, zeros, n_pad)

    return _tc_gru(partials[0, :n], partials[1, :n], h,
                   gru_w_ih.astype(jnp.bfloat16), gru_w_hh.astype(jnp.bfloat16),
                   gru_b_ih.reshape(1, 3 * hd), gru_b_hh.reshape(1, 3 * hd))


# speedup vs baseline: 5.0351x; 1.2413x over previous
import functools

import jax
import jax.numpy as jnp
from jax import lax
from jax.experimental import pallas as pl
from jax.experimental.pallas import tpu as pltpu
from jax.experimental.pallas import tpu_sc as plsc

_NC = 2
_NS = 16
_NW = _NC * _NS


def _sc_gather(table, idx):
    n, d = table.shape
    e = idx.shape[0]
    per_w = e // _NW
    ch = 80
    n_ch = per_w // ch
    assert per_w % ch == 0 and per_w * _NW == e

    mesh = plsc.VectorSubcoreMesh(core_axis_name="c", subcore_axis_name="s",
                                  num_cores=_NC, num_subcores=_NS)

    spc = 5
    scr = ch * spc
    nsc = per_w // scr
    assert per_w % scr == 0

    @functools.partial(
        pl.kernel,
        out_type=jax.ShapeDtypeStruct((e, d), jnp.float32),
        mesh=mesh,
        scratch_types=[
            pltpu.VMEM((per_w,), jnp.int32),
            pltpu.VMEM((2, scr, d), jnp.float32),
            pltpu.SemaphoreType.DMA,
            pltpu.SemaphoreType.DMA,
            pltpu.SemaphoreType.DMA,
            pltpu.SemaphoreType.DMA,
        ],
    )
    def k(table_hbm, idx_hbm, out_hbm, idx_all, bufs, g0, g1, w0, w1):
        wid = lax.axis_index("s") * _NC + lax.axis_index("c")
        base = wid * per_w
        gsem = (g0, g1)
        wsem = (w0, w1)

        pltpu.sync_copy(idx_hbm.at[pl.ds(base, per_w)], idx_all)

        wdesc = [None, None]
        for c in range(nsc):
            b = c % 2
            rows = bufs.at[b]
            if wdesc[b] is not None:
                wdesc[b].wait()
            gd = [
                pltpu.async_copy(
                    table_hbm.at[idx_all.at[pl.ds(c * scr + j * ch, ch)]],
                    rows.at[pl.ds(j * ch, ch)], gsem[b])
                for j in range(spc)
            ]
            for dsc in gd:
                dsc.wait()
            wdesc[b] = pltpu.async_copy(
                rows, out_hbm.at[pl.ds(base + c * scr, scr)], wsem[b])
        wdesc[0].wait()
        wdesc[1].wait()

    return k(table, idx)


def _sc_scatter_add(msgs, dst, zeros, n_pad):
    e, d = msgs.shape
    per_w = e // _NW
    ch = 80
    n_ch = per_w // ch
    rows_per_tile = n_pad // _NS
    assert per_w % ch == 0 and rows_per_tile % 8 == 0

    mesh = plsc.VectorSubcoreMesh(core_axis_name="c", subcore_axis_name="s",
                                  num_cores=_NC, num_subcores=_NS)

    @functools.partial(
        pl.kernel,
        out_type=jax.ShapeDtypeStruct((_NC, n_pad, d), jnp.float32),
        mesh=mesh,
        scratch_types=[
            pltpu.VMEM((per_w,), jnp.int32),
            pltpu.VMEM((2, 1, ch), jnp.int32),
            pltpu.VMEM((2, ch, d), jnp.float32),
            pltpu.VMEM_SHARED((n_pad, d), jnp.float32),
            pltpu.SemaphoreType.DMA,
            pltpu.SemaphoreType.DMA,
            pltpu.SemaphoreType.DMA,
            pltpu.SemaphoreType.DMA,
        ],
    )
    def k(msgs_hbm, dst_hbm, zeros_hbm, out_hbm, dst_all, idx_s, bufs, acc_sh,
          l0, l1, s0, s1):
        cid = lax.axis_index("c")
        sid = lax.axis_index("s")
        wid = sid * _NC + cid
        my_rows = pl.ds(sid * rows_per_tile, rows_per_tile)
        lsem = (l0, l1)
        ssem = (s0, s1)
        base = wid * per_w

        pltpu.sync_copy(zeros_hbm, acc_sh.at[my_rows])
        pltpu.sync_copy(dst_hbm.at[pl.ds(base, per_w)], dst_all)
        plsc.subcore_barrier()

        ldesc = [None, None]
        sdesc = [None, None]
        ldesc[0] = pltpu.async_copy(
            msgs_hbm.at[pl.ds(base, ch)], bufs.at[0], lsem[0])
        for c in range(n_ch):
            b = c % 2
            nb = (c + 1) % 2
            for q in range(ch // 16):
                idx_s[b, 0, pl.ds(q * 16, 16)] = (
                    dst_all[pl.ds(c * ch + q * 16, 16)])
            if c + 1 < n_ch:
                if sdesc[nb] is not None:
                    sdesc[nb].wait()
                ldesc[nb] = pltpu.async_copy(
                    msgs_hbm.at[pl.ds(base + (c + 1) * ch, ch)],
                    bufs.at[nb], lsem[nb])
            ldesc[b].wait()
            sdesc[b] = pltpu.async_copy(
                bufs.at[b], acc_sh.at[idx_s.at[b, 0]], ssem[b], add=True)
        sdesc[0].wait()
        sdesc[1].wait()
        plsc.subcore_barrier()
        pltpu.sync_copy(acc_sh.at[my_rows], out_hbm.at[cid, my_rows])

    return k(msgs, dst, zeros)


def _tc_mlp(ty_f32, h_src, edge_attr, w1h_cat, w1a_cat, b1_cat, w2_cat, b2):
    e, h = h_src.shape
    t = b2.shape[0]
    ea = edge_attr.shape[1]
    th = t * h
    be = 1280
    assert e % be == 0
    grid = (e // be,)

    def body(ty_ref, hs_ref, ea_ref, w1h_ref, w1a_ref, b1_ref, w2_ref, b2_ref,
             out_ref):
        hs = hs_ref[...].astype(jnp.bfloat16)
        eat = ea_ref[...].astype(jnp.bfloat16)
        ty = ty_ref[...]
        x1 = jnp.dot(hs, w1h_ref[...], preferred_element_type=jnp.float32)
        x1 = x1 + jnp.dot(eat, w1a_ref[...],
                          preferred_element_type=jnp.float32)
        x1 = x1 + b1_ref[...]
        x1 = jnp.maximum(x1, 0.0)
        colt = (lax.broadcasted_iota(jnp.int32, (1, th), 1) // h)
        xm = jnp.where(colt.astype(jnp.float32) == ty, x1, 0.0)
        m = jnp.dot(xm.astype(jnp.bfloat16), w2_ref[...],
                    preferred_element_type=jnp.float32)
        tyrow = lax.broadcasted_iota(jnp.int32, (1, t), 1).astype(jnp.float32)
        m4 = jnp.where(tyrow == ty, 1.0, 0.0)
        m = m + jnp.dot(m4.astype(jnp.bfloat16),
                        b2_ref[...].astype(jnp.bfloat16),
                        preferred_element_type=jnp.float32)
        out_ref[...] = m

    return pl.pallas_call(
        body,
        grid=grid,
        in_specs=[
            pl.BlockSpec((be, 1), lambda i: (i, 0)),
            pl.BlockSpec((be, h), lambda i: (i, 0)),
            pl.BlockSpec((be, ea), lambda i: (i, 0)),
            pl.BlockSpec((h, th), lambda i: (0, 0)),
            pl.BlockSpec((ea, th), lambda i: (0, 0)),
            pl.BlockSpec((1, th), lambda i: (0, 0)),
            pl.BlockSpec((th, h), lambda i: (0, 0)),
            pl.BlockSpec((t, h), lambda i: (0, 0)),
        ],
        out_specs=pl.BlockSpec((be, h), lambda i: (i, 0)),
        out_shape=jax.ShapeDtypeStruct((e, h), jnp.float32),
    )(ty_f32, h_src, edge_attr, w1h_cat, w1a_cat, b1_cat, w2_cat, b2)


def _tc_gru(msg_a, msg_b, h, w_ih, w_hh, b_ih, b_hh):
    n, hd = h.shape
    bn = 1000
    grid = (n // bn,)

    def body(ma_ref, mb_ref, h_ref, wih_ref, whh_ref, bih_ref, bhh_ref,
             out_ref):
        msg = ma_ref[...] + mb_ref[...]
        hv = h_ref[...]
        dims = (((1,), (1,)), ((), ()))
        gi = lax.dot_general(msg.astype(jnp.bfloat16), wih_ref[...], dims,
                             preferred_element_type=jnp.float32)
        gi = gi + bih_ref[...]
        gh = lax.dot_general(hv.astype(jnp.bfloat16), whh_ref[...], dims,
                             preferred_element_type=jnp.float32)
        gh = gh + bhh_ref[...]
        r = jax.nn.sigmoid(gi[:, :hd] + gh[:, :hd])
        z = jax.nn.sigmoid(gi[:, hd:2 * hd] + gh[:, hd:2 * hd])
        nn = jnp.tanh(gi[:, 2 * hd:] + r * gh[:, 2 * hd:])
        out_ref[...] = (1.0 - z) * nn + z * hv

    return pl.pallas_call(
        body,
        grid=grid,
        in_specs=[
            pl.BlockSpec((bn, hd), lambda i: (i, 0)),
            pl.BlockSpec((bn, hd), lambda i: (i, 0)),
            pl.BlockSpec((bn, hd), lambda i: (i, 0)),
            pl.BlockSpec((3 * hd, hd), lambda i: (0, 0)),
            pl.BlockSpec((3 * hd, hd), lambda i: (0, 0)),
            pl.BlockSpec((1, 3 * hd), lambda i: (0, 0)),
            pl.BlockSpec((1, 3 * hd), lambda i: (0, 0)),
        ],
        out_specs=pl.BlockSpec((bn, hd), lambda i: (i, 0)),
        out_shape=jax.ShapeDtypeStruct((n, hd), jnp.float32),
    )(msg_a, msg_b, h, w_ih, w_hh, b_ih, b_hh)


def kernel(h, edge_index, edge_type, edge_attr, W1, b1, W2, b2,
           gru_w_ih, gru_w_hh, gru_b_ih, gru_b_hh):
    n, hd = h.shape
    e = edge_index.shape[1]

    src = edge_index[0].astype(jnp.int32)
    dst = edge_index[1].astype(jnp.int32)
    ty_f32 = edge_type.astype(jnp.float32).reshape(e, 1)

    h_src = _sc_gather(h, src)

    t = W1.shape[0]
    th = t * hd
    w1h_cat = W1[:, :hd, :].transpose(1, 0, 2).reshape(hd, th)
    w1a_cat = W1[:, hd:, :].transpose(1, 0, 2).reshape(-1, th)
    b1_cat = b1.reshape(1, th)
    w2_cat = W2.reshape(th, hd)
    msgs = _tc_mlp(ty_f32, h_src, edge_attr,
                   w1h_cat.astype(jnp.bfloat16), w1a_cat.astype(jnp.bfloat16),
                   b1_cat, w2_cat.astype(jnp.bfloat16), b2)

    n_pad = ((n + 8 * _NS - 1) // (8 * _NS)) * (8 * _NS)
    zeros = jnp.zeros((n_pad // _NS, hd), jnp.float32)
    partials = _sc_scatter_add(msgs, dst, zeros, n_pad)

    return _tc_gru(partials[0, :n], partials[1, :n], h,
                   gru_w_ih.astype(jnp.bfloat16), gru_w_hh.astype(jnp.bfloat16),
                   gru_b_ih.reshape(1, 3 * hd), gru_b_hh.reshape(1, 3 * hd))


# speedup vs baseline: 5.9296x; 1.1777x over previous
import functools

import jax
import jax.numpy as jnp
from jax import lax
from jax.experimental import pallas as pl
from jax.experimental.pallas import tpu as pltpu
from jax.experimental.pallas import tpu_sc as plsc

_NC = 2
_NS = 16
_NW = _NC * _NS


def _sc_gather(table, idx):
    n, d = table.shape
    e = idx.shape[0]
    per_w = e // _NW
    ch = 80
    n_ch = per_w // ch
    assert per_w % ch == 0 and per_w * _NW == e

    mesh = plsc.VectorSubcoreMesh(core_axis_name="c", subcore_axis_name="s",
                                  num_cores=_NC, num_subcores=_NS)

    spc = 5
    scr = ch * spc
    nsc = per_w // scr
    assert per_w % scr == 0

    @functools.partial(
        pl.kernel,
        out_type=jax.ShapeDtypeStruct((e, d), jnp.float32),
        mesh=mesh,
        scratch_types=[
            pltpu.VMEM((per_w,), jnp.int32),
            pltpu.VMEM((2, scr, d), jnp.float32),
            pltpu.SemaphoreType.DMA,
            pltpu.SemaphoreType.DMA,
            pltpu.SemaphoreType.DMA,
            pltpu.SemaphoreType.DMA,
        ],
    )
    def k(table_hbm, idx_hbm, out_hbm, idx_all, bufs, g0, g1, w0, w1):
        wid = lax.axis_index("s") * _NC + lax.axis_index("c")
        base = wid * per_w
        gsem = (g0, g1)
        wsem = (w0, w1)

        pltpu.sync_copy(idx_hbm.at[pl.ds(base, per_w)], idx_all)

        wdesc = [None, None]
        for c in range(nsc):
            b = c % 2
            rows = bufs.at[b]
            if wdesc[b] is not None:
                wdesc[b].wait()
            gd = [
                pltpu.async_copy(
                    table_hbm.at[idx_all.at[pl.ds(c * scr + j * ch, ch)]],
                    rows.at[pl.ds(j * ch, ch)], gsem[b])
                for j in range(spc)
            ]
            for dsc in gd:
                dsc.wait()
            wdesc[b] = pltpu.async_copy(
                rows, out_hbm.at[pl.ds(base + c * scr, scr)], wsem[b])
        wdesc[0].wait()
        wdesc[1].wait()

    return k(table, idx)


def _sc_scatter_add(msgs, dst, zeros, n_pad):
    e, d = msgs.shape
    per_w = e // _NW
    ch = 80
    n_ch = per_w // ch
    rows_per_tile = n_pad // _NS
    assert per_w % ch == 0 and rows_per_tile % 8 == 0

    mesh = plsc.VectorSubcoreMesh(core_axis_name="c", subcore_axis_name="s",
                                  num_cores=_NC, num_subcores=_NS)

    @functools.partial(
        pl.kernel,
        out_type=jax.ShapeDtypeStruct((_NC, n_pad, d), jnp.float32),
        mesh=mesh,
        scratch_types=[
            pltpu.VMEM((per_w,), jnp.int32),
            pltpu.VMEM((2, 1, ch), jnp.int32),
            pltpu.VMEM((2, ch, d), jnp.float32),
            pltpu.VMEM_SHARED((n_pad, d), jnp.float32),
            pltpu.SemaphoreType.DMA,
            pltpu.SemaphoreType.DMA,
            pltpu.SemaphoreType.DMA,
            pltpu.SemaphoreType.DMA,
        ],
    )
    def k(msgs_hbm, dst_hbm, zeros_hbm, out_hbm, dst_all, idx_s, bufs, acc_sh,
          l0, l1, s0, s1):
        cid = lax.axis_index("c")
        sid = lax.axis_index("s")
        wid = sid * _NC + cid
        my_rows = pl.ds(sid * rows_per_tile, rows_per_tile)
        lsem = (l0, l1)
        ssem = (s0, s1)
        base = wid * per_w

        pltpu.sync_copy(zeros_hbm, acc_sh.at[my_rows])
        pltpu.sync_copy(dst_hbm.at[pl.ds(base, per_w)], dst_all)
        plsc.subcore_barrier()

        ldesc = [None, None]
        sdesc = [None, None]
        ldesc[0] = pltpu.async_copy(
            msgs_hbm.at[pl.ds(base, ch)], bufs.at[0], lsem[0])
        for c in range(n_ch):
            b = c % 2
            nb = (c + 1) % 2
            for q in range(ch // 16):
                idx_s[b, 0, pl.ds(q * 16, 16)] = (
                    dst_all[pl.ds(c * ch + q * 16, 16)])
            if c + 1 < n_ch:
                if sdesc[nb] is not None:
                    sdesc[nb].wait()
                ldesc[nb] = pltpu.async_copy(
                    msgs_hbm.at[pl.ds(base + (c + 1) * ch, ch)],
                    bufs.at[nb], lsem[nb])
            ldesc[b].wait()
            sdesc[b] = pltpu.async_copy(
                bufs.at[b], acc_sh.at[idx_s.at[b, 0]], ssem[b], add=True)
        sdesc[0].wait()
        sdesc[1].wait()
        plsc.subcore_barrier()
        pltpu.sync_copy(acc_sh.at[my_rows], out_hbm.at[cid, my_rows])

    return k(msgs, dst, zeros)


def _tc_mlp(ty_row, h_src, eaT_aug, w1h_cat, w1a_aug, w2_cat, expand4, b2):
    e, h = h_src.shape
    t, th = expand4.shape[0], expand4.shape[1]
    ea1 = eaT_aug.shape[0]
    be = 2560
    assert e % be == 0
    grid = (e // be,)

    def body(ty_ref, hs_ref, ea_ref, w1h_ref, w1a_ref, w2_ref, ex4_ref,
             b2_ref, out_ref):
        hs = hs_ref[...].astype(jnp.bfloat16)
        eat = ea_ref[...].astype(jnp.bfloat16)
        ty = ty_ref[...]
        cd0 = (((0,), (0,)), ((), ()))
        x1 = jnp.dot(hs, w1h_ref[...], preferred_element_type=jnp.float32)
        x1 = x1 + lax.dot_general(eat, w1a_ref[...], cd0,
                                  preferred_element_type=jnp.float32)
        x1 = jnp.maximum(x1, 0.0)
        tycol = lax.broadcasted_iota(jnp.int32, (t, 1), 0).astype(jnp.float32)
        m4 = jnp.where(ty == tycol, 1.0, 0.0).astype(jnp.bfloat16)
        mask = lax.dot_general(m4, ex4_ref[...], cd0,
                               preferred_element_type=jnp.float32)
        xm = (x1 * mask).astype(jnp.bfloat16)
        m = jnp.dot(xm, w2_ref[...], preferred_element_type=jnp.float32)
        m = m + lax.dot_general(m4, b2_ref[...].astype(jnp.bfloat16), cd0,
                                preferred_element_type=jnp.float32)
        out_ref[...] = m

    return pl.pallas_call(
        body,
        grid=grid,
        in_specs=[
            pl.BlockSpec((1, be), lambda i: (0, i)),
            pl.BlockSpec((be, h), lambda i: (i, 0)),
            pl.BlockSpec((ea1, be), lambda i: (0, i)),
            pl.BlockSpec((h, th), lambda i: (0, 0)),
            pl.BlockSpec((ea1, th), lambda i: (0, 0)),
            pl.BlockSpec((th, h), lambda i: (0, 0)),
            pl.BlockSpec((t, th), lambda i: (0, 0)),
            pl.BlockSpec((t, h), lambda i: (0, 0)),
        ],
        out_specs=pl.BlockSpec((be, h), lambda i: (i, 0)),
        out_shape=jax.ShapeDtypeStruct((e, h), jnp.float32),
    )(ty_row, h_src, eaT_aug, w1h_cat, w1a_aug, w2_cat, expand4, b2)


def _tc_gru(msg_a, msg_b, h, w_ih, w_hh, b_ih, b_hh):
    n, hd = h.shape
    bn = 1000
    grid = (n // bn,)

    def body(ma_ref, mb_ref, h_ref, wih_ref, whh_ref, bih_ref, bhh_ref,
             out_ref):
        msg = ma_ref[...] + mb_ref[...]
        hv = h_ref[...]
        dims = (((1,), (1,)), ((), ()))
        gi = lax.dot_general(msg.astype(jnp.bfloat16), wih_ref[...], dims,
                             preferred_element_type=jnp.float32)
        gi = gi + bih_ref[...]
        gh = lax.dot_general(hv.astype(jnp.bfloat16), whh_ref[...], dims,
                             preferred_element_type=jnp.float32)
        gh = gh + bhh_ref[...]
        r = jax.nn.sigmoid(gi[:, :hd] + gh[:, :hd])
        z = jax.nn.sigmoid(gi[:, hd:2 * hd] + gh[:, hd:2 * hd])
        nn = jnp.tanh(gi[:, 2 * hd:] + r * gh[:, 2 * hd:])
        out_ref[...] = (1.0 - z) * nn + z * hv

    return pl.pallas_call(
        body,
        grid=grid,
        in_specs=[
            pl.BlockSpec((bn, hd), lambda i: (i, 0)),
            pl.BlockSpec((bn, hd), lambda i: (i, 0)),
            pl.BlockSpec((bn, hd), lambda i: (i, 0)),
            pl.BlockSpec((3 * hd, hd), lambda i: (0, 0)),
            pl.BlockSpec((3 * hd, hd), lambda i: (0, 0)),
            pl.BlockSpec((1, 3 * hd), lambda i: (0, 0)),
            pl.BlockSpec((1, 3 * hd), lambda i: (0, 0)),
        ],
        out_specs=pl.BlockSpec((bn, hd), lambda i: (i, 0)),
        out_shape=jax.ShapeDtypeStruct((n, hd), jnp.float32),
    )(msg_a, msg_b, h, w_ih, w_hh, b_ih, b_hh)


def kernel(h, edge_index, edge_type, edge_attr, W1, b1, W2, b2,
           gru_w_ih, gru_w_hh, gru_b_ih, gru_b_hh):
    n, hd = h.shape
    e = edge_index.shape[1]

    src = edge_index[0].astype(jnp.int32)
    dst = edge_index[1].astype(jnp.int32)
    ty_row = edge_type.astype(jnp.float32).reshape(1, e)

    h_src = _sc_gather(h, src)

    t = W1.shape[0]
    th = t * hd
    w1h_cat = W1[:, :hd, :].transpose(1, 0, 2).reshape(hd, th)
    w1a_cat = W1[:, hd:, :].transpose(1, 0, 2).reshape(-1, th)
    b1_cat = b1.reshape(1, th)
    w1a_aug = jnp.concatenate([w1a_cat, b1_cat], axis=0)
    eaT_aug = jnp.concatenate(
        [edge_attr.T, jnp.ones((1, e), jnp.float32)], axis=0)
    w2_cat = W2.reshape(th, hd)
    expand4 = jnp.repeat(jnp.eye(t, dtype=jnp.bfloat16), hd, axis=1)
    msgs = _tc_mlp(ty_row, h_src, eaT_aug,
                   w1h_cat.astype(jnp.bfloat16), w1a_aug.astype(jnp.bfloat16),
                   w2_cat.astype(jnp.bfloat16), expand4, b2)

    n_pad = ((n + 8 * _NS - 1) // (8 * _NS)) * (8 * _NS)
    zeros = jnp.zeros((n_pad // _NS, hd), jnp.float32)
    partials = _sc_scatter_add(msgs, dst, zeros, n_pad)

    return _tc_gru(partials[0, :n], partials[1, :n], h,
                   gru_w_ih.astype(jnp.bfloat16), gru_w_hh.astype(jnp.bfloat16),
                   gru_b_ih.reshape(1, 3 * hd), gru_b_hh.reshape(1, 3 * hd))


# speedup vs baseline: 6.7617x; 1.1403x over previous
import functools

import jax
import jax.numpy as jnp
from jax import lax
from jax.experimental import pallas as pl
from jax.experimental.pallas import tpu as pltpu
from jax.experimental.pallas import tpu_sc as plsc

_NC = 2
_NS = 16
_NW = _NC * _NS


def _sc_gather(table, idx):
    n, d = table.shape
    e = idx.shape[0]
    per_w = e // _NW
    ch = 80
    n_ch = per_w // ch
    assert per_w % ch == 0 and per_w * _NW == e

    mesh = plsc.VectorSubcoreMesh(core_axis_name="c", subcore_axis_name="s",
                                  num_cores=_NC, num_subcores=_NS)

    spc = next(s for s in (5, 4, 3, 2, 1) if n_ch % s == 0)
    scr = ch * spc
    nsc = per_w // scr
    assert per_w % scr == 0

    @functools.partial(
        pl.kernel,
        out_type=jax.ShapeDtypeStruct((e, d), jnp.float32),
        mesh=mesh,
        scratch_types=[
            pltpu.VMEM((per_w,), jnp.int32),
            pltpu.VMEM((2, scr, d), jnp.float32),
            pltpu.SemaphoreType.DMA,
            pltpu.SemaphoreType.DMA,
            pltpu.SemaphoreType.DMA,
            pltpu.SemaphoreType.DMA,
        ],
    )
    def k(table_hbm, idx_hbm, out_hbm, idx_all, bufs, g0, g1, w0, w1):
        wid = lax.axis_index("s") * _NC + lax.axis_index("c")
        base = wid * per_w
        gsem = (g0, g1)
        wsem = (w0, w1)

        pltpu.sync_copy(idx_hbm.at[pl.ds(base, per_w)], idx_all)

        wdesc = [None, None]
        for c in range(nsc):
            b = c % 2
            rows = bufs.at[b]
            if wdesc[b] is not None:
                wdesc[b].wait()
            gd = [
                pltpu.async_copy(
                    table_hbm.at[idx_all.at[pl.ds(c * scr + j * ch, ch)]],
                    rows.at[pl.ds(j * ch, ch)], gsem[b])
                for j in range(spc)
            ]
            for dsc in gd:
                dsc.wait()
            wdesc[b] = pltpu.async_copy(
                rows, out_hbm.at[pl.ds(base + c * scr, scr)], wsem[b])
        wdesc[0].wait()
        wdesc[1].wait()

    return k(table, idx)


def _sc_scatter_add(msgs, dst, zeros, n_pad):
    e, d = msgs.shape
    per_w = e // _NW
    ch = 80
    n_ch = per_w // ch
    rows_per_tile = n_pad // _NS
    assert per_w % ch == 0 and rows_per_tile % 8 == 0

    mesh = plsc.VectorSubcoreMesh(core_axis_name="c", subcore_axis_name="s",
                                  num_cores=_NC, num_subcores=_NS)

    @functools.partial(
        pl.kernel,
        out_type=jax.ShapeDtypeStruct((_NC, n_pad, d), jnp.float32),
        mesh=mesh,
        scratch_types=[
            pltpu.VMEM((per_w,), jnp.int32),
            pltpu.VMEM((2, 1, ch), jnp.int32),
            pltpu.VMEM((2, ch, d), jnp.float32),
            pltpu.VMEM_SHARED((n_pad, d), jnp.float32),
            pltpu.SemaphoreType.DMA,
            pltpu.SemaphoreType.DMA,
            pltpu.SemaphoreType.DMA,
            pltpu.SemaphoreType.DMA,
        ],
    )
    def k(msgs_hbm, dst_hbm, zeros_hbm, out_hbm, dst_all, idx_s, bufs, acc_sh,
          l0, l1, s0, s1):
        cid = lax.axis_index("c")
        sid = lax.axis_index("s")
        wid = sid * _NC + cid
        my_rows = pl.ds(sid * rows_per_tile, rows_per_tile)
        lsem = (l0, l1)
        ssem = (s0, s1)
        base = wid * per_w

        pltpu.sync_copy(zeros_hbm, acc_sh.at[my_rows])
        pltpu.sync_copy(dst_hbm.at[pl.ds(base, per_w)], dst_all)
        plsc.subcore_barrier()

        ldesc = [None, None]
        sdesc = [None, None]
        ldesc[0] = pltpu.async_copy(
            msgs_hbm.at[pl.ds(base, ch)], bufs.at[0], lsem[0])
        for c in range(n_ch):
            b = c % 2
            nb = (c + 1) % 2
            for q in range(ch // 16):
                idx_s[b, 0, pl.ds(q * 16, 16)] = (
                    dst_all[pl.ds(c * ch + q * 16, 16)])
            if c + 1 < n_ch:
                if sdesc[nb] is not None:
                    sdesc[nb].wait()
                ldesc[nb] = pltpu.async_copy(
                    msgs_hbm.at[pl.ds(base + (c + 1) * ch, ch)],
                    bufs.at[nb], lsem[nb])
            ldesc[b].wait()
            sdesc[b] = pltpu.async_copy(
                bufs.at[b], acc_sh.at[idx_s.at[b, 0]], ssem[b], add=True)
        sdesc[0].wait()
        sdesc[1].wait()
        plsc.subcore_barrier()
        pltpu.sync_copy(acc_sh.at[my_rows], out_hbm.at[cid, my_rows])

    return k(msgs, dst, zeros)


def _tc_mlp(ty_row, h_src, eaT_aug, w1h_cat, w1a_aug, w2_cat, expand4, b2):
    e, h = h_src.shape
    t = b2.shape[0]
    th = w2_cat.shape[0]
    ea1 = eaT_aug.shape[0]
    be = 2560
    assert e % be == 0
    grid = (e // be,)

    def body(ty_ref, hs_ref, ea_ref, w1h_ref, w1a_ref, w2_ref, ex4_ref,
             b2_ref, out_ref):
        hs = hs_ref[...].astype(jnp.bfloat16)
        eat = ea_ref[...].astype(jnp.bfloat16)
        ty = ty_ref[...]
        cd0 = (((0,), (0,)), ((), ()))
        x1 = jnp.dot(hs, w1h_ref[...], preferred_element_type=jnp.float32)
        x1 = x1 + lax.dot_general(eat, w1a_ref[...], cd0,
                                  preferred_element_type=jnp.float32)
        x1 = jnp.maximum(x1, 0.0)
        tycol = lax.broadcasted_iota(jnp.int32, (t, 1), 0).astype(jnp.float32)
        m4 = jnp.where(ty == tycol, 1.0, 0.0).astype(jnp.bfloat16)
        mask = lax.dot_general(m4, ex4_ref[...], cd0,
                               preferred_element_type=jnp.float32)
        xm = (x1 * mask).astype(jnp.bfloat16)
        m = jnp.dot(xm, w2_ref[...], preferred_element_type=jnp.float32)
        m = m + lax.dot_general(m4, b2_ref[...].astype(jnp.bfloat16), cd0,
                                preferred_element_type=jnp.float32)
        out_ref[...] = m

    return pl.pallas_call(
        body,
        grid=grid,
        in_specs=[
            pl.BlockSpec((1, be), lambda i: (0, i)),
            pl.BlockSpec((be, h), lambda i: (i, 0)),
            pl.BlockSpec((ea1, be), lambda i: (0, i)),
            pl.BlockSpec((h, th), lambda i: (0, 0)),
            pl.BlockSpec((ea1, th), lambda i: (0, 0)),
            pl.BlockSpec((th, h), lambda i: (0, 0)),
            pl.BlockSpec((t, th), lambda i: (0, 0)),
            pl.BlockSpec((t, h), lambda i: (0, 0)),
        ],
        out_specs=pl.BlockSpec((be, h), lambda i: (i, 0)),
        out_shape=jax.ShapeDtypeStruct((e, h), jnp.float32),
    )(ty_row, h_src, eaT_aug, w1h_cat, w1a_aug, w2_cat, expand4, b2)


def _tc_gru(msgs_parts, h, w_ih, w_hh, b_ih, b_hh):
    n, hd = h.shape
    bn = 1000
    grid = (n // bn,)
    np_ = len(msgs_parts)

    def body(*refs):
        part_refs = refs[:np_]
        h_ref, wih_ref, whh_ref, bih_ref, bhh_ref, out_ref = refs[np_:]
        msg = part_refs[0][...]
        for pr in part_refs[1:]:
            msg = msg + pr[...]
        hv = h_ref[...]
        dims = (((1,), (1,)), ((), ()))
        gi = lax.dot_general(msg.astype(jnp.bfloat16), wih_ref[...], dims,
                             preferred_element_type=jnp.float32)
        gi = gi + bih_ref[...]
        gh = lax.dot_general(hv.astype(jnp.bfloat16), whh_ref[...], dims,
                             preferred_element_type=jnp.float32)
        gh = gh + bhh_ref[...]
        r = jax.nn.sigmoid(gi[:, :hd] + gh[:, :hd])
        z = jax.nn.sigmoid(gi[:, hd:2 * hd] + gh[:, hd:2 * hd])
        nn = jnp.tanh(gi[:, 2 * hd:] + r * gh[:, 2 * hd:])
        out_ref[...] = (1.0 - z) * nn + z * hv

    return pl.pallas_call(
        body,
        grid=grid,
        in_specs=(
            [pl.BlockSpec((bn, hd), lambda i: (i, 0)) for _ in range(np_)]
            + [
                pl.BlockSpec((bn, hd), lambda i: (i, 0)),
                pl.BlockSpec((3 * hd, hd), lambda i: (0, 0)),
                pl.BlockSpec((3 * hd, hd), lambda i: (0, 0)),
                pl.BlockSpec((1, 3 * hd), lambda i: (0, 0)),
                pl.BlockSpec((1, 3 * hd), lambda i: (0, 0)),
            ]
        ),
        out_specs=pl.BlockSpec((bn, hd), lambda i: (i, 0)),
        out_shape=jax.ShapeDtypeStruct((n, hd), jnp.float32),
    )(*msgs_parts, h, w_ih, w_hh, b_ih, b_hh)


def kernel(h, edge_index, edge_type, edge_attr, W1, b1, W2, b2,
           gru_w_ih, gru_w_hh, gru_b_ih, gru_b_hh):
    n, hd = h.shape
    e = edge_index.shape[1]

    src = edge_index[0].astype(jnp.int32)
    dst = edge_index[1].astype(jnp.int32)
    ty_row = edge_type.astype(jnp.float32).reshape(1, e)

    t = W1.shape[0]
    th = t * hd
    w1h_cat = W1[:, :hd, :].transpose(1, 0, 2).reshape(hd, th)
    w1a_cat = W1[:, hd:, :].transpose(1, 0, 2).reshape(-1, th)
    b1_cat = b1.reshape(1, th)
    w1a_aug = jnp.concatenate([w1a_cat, b1_cat], axis=0)
    eaT_aug = jnp.concatenate(
        [edge_attr.T, jnp.ones((1, e), jnp.float32)], axis=0)
    w2_cat = W2.reshape(th, hd)
    expand4 = jnp.repeat(jnp.eye(t, dtype=jnp.bfloat16), hd, axis=1)
    w1h_bf = w1h_cat.astype(jnp.bfloat16)
    w1a_bf = w1a_aug.astype(jnp.bfloat16)
    w2_bf = w2_cat.astype(jnp.bfloat16)

    n_pad = ((n + 8 * _NS - 1) // (8 * _NS)) * (8 * _NS)
    zeros = jnp.zeros((n_pad // _NS, hd), jnp.float32)

    e_a = (e // 2560 // 2) * 2560
    partials = []
    msgs_halves = []
    for lo, hi in ((0, e_a), (e_a, e)):
        h_src = _sc_gather(h, lax.slice(src, (lo,), (hi,)))
        msgs = _tc_mlp(lax.slice(ty_row, (0, lo), (1, hi)), h_src,
                       lax.slice(eaT_aug, (0, lo), (eaT_aug.shape[0], hi)),
                       w1h_bf, w1a_bf, w2_bf, expand4, b2)
        msgs_halves.append((msgs, lax.slice(dst, (lo,), (hi,))))
    for msgs, dst_half in msgs_halves:
        p = _sc_scatter_add(msgs, dst_half, zeros, n_pad)
        partials.extend([p[0, :n], p[1, :n]])

    return _tc_gru(partials, h,
                   gru_w_ih.astype(jnp.bfloat16), gru_w_hh.astype(jnp.bfloat16),
                   gru_b_ih.reshape(1, 3 * hd), gru_b_hh.reshape(1, 3 * hd))


# speedup vs baseline: 6.9527x; 1.0282x over previous
import functools

import jax
import jax.numpy as jnp
from jax import lax
from jax.experimental import pallas as pl
from jax.experimental.pallas import tpu as pltpu
from jax.experimental.pallas import tpu_sc as plsc

_NC = 2
_NS = 16
_NW = _NC * _NS


def _sc_gather(table, idx):
    n, d = table.shape
    e = idx.shape[0]
    per_w = e // _NW
    ch = 80
    n_ch = per_w // ch
    assert per_w % ch == 0 and per_w * _NW == e

    mesh = plsc.VectorSubcoreMesh(core_axis_name="c", subcore_axis_name="s",
                                  num_cores=_NC, num_subcores=_NS)

    spc = next(s for s in (5, 4, 3, 2, 1) if n_ch % s == 0)
    scr = ch * spc
    nsc = per_w // scr
    assert per_w % scr == 0

    @functools.partial(
        pl.kernel,
        out_type=jax.ShapeDtypeStruct((e, d), jnp.float32),
        mesh=mesh,
        scratch_types=[
            pltpu.VMEM((per_w,), jnp.int32),
            pltpu.VMEM((2, scr, d), jnp.float32),
            pltpu.SemaphoreType.DMA,
            pltpu.SemaphoreType.DMA,
            pltpu.SemaphoreType.DMA,
            pltpu.SemaphoreType.DMA,
        ],
    )
    def k(table_hbm, idx_hbm, out_hbm, idx_all, bufs, g0, g1, w0, w1):
        wid = lax.axis_index("s") * _NC + lax.axis_index("c")
        base = wid * per_w
        gsem = (g0, g1)
        wsem = (w0, w1)

        pltpu.sync_copy(idx_hbm.at[pl.ds(base, per_w)], idx_all)

        wdesc = [None, None]
        for c in range(nsc):
            b = c % 2
            rows = bufs.at[b]
            if wdesc[b] is not None:
                wdesc[b].wait()
            gd = [
                pltpu.async_copy(
                    table_hbm.at[idx_all.at[pl.ds(c * scr + j * ch, ch)]],
                    rows.at[pl.ds(j * ch, ch)], gsem[b])
                for j in range(spc)
            ]
            for dsc in gd:
                dsc.wait()
            wdesc[b] = pltpu.async_copy(
                rows, out_hbm.at[pl.ds(base + c * scr, scr)], wsem[b])
        wdesc[0].wait()
        wdesc[1].wait()

    return k(table, idx)


def _sc_scatter_add(msgs, dst, zeros, n_pad):
    e, d = msgs.shape
    per_w = e // _NW
    ch = 80
    n_ch = per_w // ch
    rows_per_tile = n_pad // _NS
    assert per_w % ch == 0 and rows_per_tile % 8 == 0

    mesh = plsc.VectorSubcoreMesh(core_axis_name="c", subcore_axis_name="s",
                                  num_cores=_NC, num_subcores=_NS)

    @functools.partial(
        pl.kernel,
        out_type=jax.ShapeDtypeStruct((_NC, n_pad, d), jnp.float32),
        mesh=mesh,
        scratch_types=[
            pltpu.VMEM((per_w,), jnp.int32),
            pltpu.VMEM((2, 1, ch), jnp.int32),
            pltpu.VMEM((2, ch, d), jnp.float32),
            pltpu.VMEM_SHARED((n_pad, d), jnp.float32),
            pltpu.SemaphoreType.DMA,
            pltpu.SemaphoreType.DMA,
            pltpu.SemaphoreType.DMA,
            pltpu.SemaphoreType.DMA,
        ],
    )
    def k(msgs_hbm, dst_hbm, zeros_hbm, out_hbm, dst_all, idx_s, bufs, acc_sh,
          l0, l1, s0, s1):
        cid = lax.axis_index("c")
        sid = lax.axis_index("s")
        wid = sid * _NC + cid
        my_rows = pl.ds(sid * rows_per_tile, rows_per_tile)
        lsem = (l0, l1)
        ssem = (s0, s1)
        base = wid * per_w

        pltpu.sync_copy(zeros_hbm, acc_sh.at[my_rows])
        pltpu.sync_copy(dst_hbm.at[pl.ds(base, per_w)], dst_all)
        plsc.subcore_barrier()

        ldesc = [None, None]
        sdesc = [None, None]
        ldesc[0] = pltpu.async_copy(
            msgs_hbm.at[pl.ds(base, ch)], bufs.at[0], lsem[0])
        for c in range(n_ch):
            b = c % 2
            nb = (c + 1) % 2
            for q in range(ch // 16):
                idx_s[b, 0, pl.ds(q * 16, 16)] = (
                    dst_all[pl.ds(c * ch + q * 16, 16)])
            if c + 1 < n_ch:
                if sdesc[nb] is not None:
                    sdesc[nb].wait()
                ldesc[nb] = pltpu.async_copy(
                    msgs_hbm.at[pl.ds(base + (c + 1) * ch, ch)],
                    bufs.at[nb], lsem[nb])
            ldesc[b].wait()
            sdesc[b] = pltpu.async_copy(
                bufs.at[b], acc_sh.at[idx_s.at[b, 0]], ssem[b], add=True)
        sdesc[0].wait()
        sdesc[1].wait()
        plsc.subcore_barrier()
        pltpu.sync_copy(acc_sh.at[my_rows], out_hbm.at[cid, my_rows])

    return k(msgs, dst, zeros)


def _tc_mlp(ty_row, h_src, eaT_aug, w1h_cat, w1a_aug, w2_cat, expand4, b2):
    e, h = h_src.shape
    t = b2.shape[0]
    th = w2_cat.shape[0]
    ea1 = eaT_aug.shape[0]
    be = 2560
    assert e % be == 0
    grid = (e // be,)

    def body(ty_ref, hs_ref, ea_ref, w1h_ref, w1a_ref, w2_ref, ex4_ref,
             b2_ref, out_ref):
        hs = hs_ref[...].astype(jnp.bfloat16)
        eat = ea_ref[...].astype(jnp.bfloat16)
        ty = ty_ref[...]
        cd0 = (((0,), (0,)), ((), ()))
        x1 = jnp.dot(hs, w1h_ref[...], preferred_element_type=jnp.float32)
        x1 = x1 + lax.dot_general(eat, w1a_ref[...], cd0,
                                  preferred_element_type=jnp.float32)
        x1 = jnp.maximum(x1, 0.0)
        tycol = lax.broadcasted_iota(jnp.int32, (t, 1), 0).astype(jnp.float32)
        m4 = jnp.where(ty == tycol, 1.0, 0.0).astype(jnp.bfloat16)
        mask = lax.dot_general(m4, ex4_ref[...], cd0,
                               preferred_element_type=jnp.float32)
        xm = (x1 * mask).astype(jnp.bfloat16)
        m = jnp.dot(xm, w2_ref[...], preferred_element_type=jnp.float32)
        m = m + lax.dot_general(m4, b2_ref[...].astype(jnp.bfloat16), cd0,
                                preferred_element_type=jnp.float32)
        out_ref[...] = m

    return pl.pallas_call(
        body,
        grid=grid,
        in_specs=[
            pl.BlockSpec((1, be), lambda i: (0, i)),
            pl.BlockSpec((be, h), lambda i: (i, 0)),
            pl.BlockSpec((ea1, be), lambda i: (0, i)),
            pl.BlockSpec((h, th), lambda i: (0, 0)),
            pl.BlockSpec((ea1, th), lambda i: (0, 0)),
            pl.BlockSpec((th, h), lambda i: (0, 0)),
            pl.BlockSpec((t, th), lambda i: (0, 0)),
            pl.BlockSpec((t, h), lambda i: (0, 0)),
        ],
        out_specs=pl.BlockSpec((be, h), lambda i: (i, 0)),
        out_shape=jax.ShapeDtypeStruct((e, h), jnp.float32),
    )(ty_row, h_src, eaT_aug, w1h_cat, w1a_aug, w2_cat, expand4, b2)


def _tc_gru(msgs_parts, h, w_ih, w_hh, b_ih, b_hh):
    n, hd = h.shape
    bn = 1000
    grid = (n // bn,)
    np_ = len(msgs_parts)

    def body(*refs):
        part_refs = refs[:np_]
        h_ref, wih_ref, whh_ref, bih_ref, bhh_ref, out_ref = refs[np_:]
        msg = part_refs[0][...]
        for pr in part_refs[1:]:
            msg = msg + pr[...]
        hv = h_ref[...]
        dims = (((1,), (1,)), ((), ()))
        gi = lax.dot_general(msg.astype(jnp.bfloat16), wih_ref[...], dims,
                             preferred_element_type=jnp.float32)
        gi = gi + bih_ref[...]
        gh = lax.dot_general(hv.astype(jnp.bfloat16), whh_ref[...], dims,
                             preferred_element_type=jnp.float32)
        gh = gh + bhh_ref[...]
        r = jax.nn.sigmoid(gi[:, :hd] + gh[:, :hd])
        z = jax.nn.sigmoid(gi[:, hd:2 * hd] + gh[:, hd:2 * hd])
        nn = jnp.tanh(gi[:, 2 * hd:] + r * gh[:, 2 * hd:])
        out_ref[...] = (1.0 - z) * nn + z * hv

    return pl.pallas_call(
        body,
        grid=grid,
        in_specs=(
            [pl.BlockSpec((bn, hd), lambda i: (i, 0)) for _ in range(np_)]
            + [
                pl.BlockSpec((bn, hd), lambda i: (i, 0)),
                pl.BlockSpec((3 * hd, hd), lambda i: (0, 0)),
                pl.BlockSpec((3 * hd, hd), lambda i: (0, 0)),
                pl.BlockSpec((1, 3 * hd), lambda i: (0, 0)),
                pl.BlockSpec((1, 3 * hd), lambda i: (0, 0)),
            ]
        ),
        out_specs=pl.BlockSpec((bn, hd), lambda i: (i, 0)),
        out_shape=jax.ShapeDtypeStruct((n, hd), jnp.float32),
    )(*msgs_parts, h, w_ih, w_hh, b_ih, b_hh)


def kernel(h, edge_index, edge_type, edge_attr, W1, b1, W2, b2,
           gru_w_ih, gru_w_hh, gru_b_ih, gru_b_hh):
    n, hd = h.shape
    e = edge_index.shape[1]

    src = edge_index[0].astype(jnp.int32)
    dst = edge_index[1].astype(jnp.int32)
    ty_row = edge_type.astype(jnp.float32).reshape(1, e)

    t = W1.shape[0]
    th = t * hd
    w1h_cat = W1[:, :hd, :].transpose(1, 0, 2).reshape(hd, th)
    w1a_cat = W1[:, hd:, :].transpose(1, 0, 2).reshape(-1, th)
    b1_cat = b1.reshape(1, th)
    w1a_aug = jnp.concatenate([w1a_cat, b1_cat], axis=0)
    eaT_aug = jnp.concatenate(
        [edge_attr.T, jnp.ones((1, e), jnp.float32)], axis=0)
    w2_cat = W2.reshape(th, hd)
    expand4 = jnp.repeat(jnp.eye(t, dtype=jnp.bfloat16), hd, axis=1)
    w1h_bf = w1h_cat.astype(jnp.bfloat16)
    w1a_bf = w1a_aug.astype(jnp.bfloat16)
    w2_bf = w2_cat.astype(jnp.bfloat16)

    n_pad = ((n + 8 * _NS - 1) // (8 * _NS)) * (8 * _NS)
    zeros = jnp.zeros((n_pad // _NS, hd), jnp.float32)

    nblk = e // 2560
    cuts = [0] + [(nblk * q // 4) * 2560 for q in (1, 2, 3)] + [e]
    partials = []
    msgs_halves = []
    for lo, hi in zip(cuts[:-1], cuts[1:]):
        h_src = _sc_gather(h, lax.slice(src, (lo,), (hi,)))
        msgs = _tc_mlp(lax.slice(ty_row, (0, lo), (1, hi)), h_src,
                       lax.slice(eaT_aug, (0, lo), (eaT_aug.shape[0], hi)),
                       w1h_bf, w1a_bf, w2_bf, expand4, b2)
        msgs_halves.append((msgs, lax.slice(dst, (lo,), (hi,))))
    for msgs, dst_half in msgs_halves:
        p = _sc_scatter_add(msgs, dst_half, zeros, n_pad)
        partials.extend([p[0, :n], p[1, :n]])

    return _tc_gru(partials, h,
                   gru_w_ih.astype(jnp.bfloat16), gru_w_hh.astype(jnp.bfloat16),
                   gru_b_ih.reshape(1, 3 * hd), gru_b_hh.reshape(1, 3 * hd))


# speedup vs baseline: 7.2253x; 1.0392x over previous
import functools

import jax
import jax.numpy as jnp
from jax import lax
from jax.experimental import pallas as pl
from jax.experimental.pallas import tpu as pltpu
from jax.experimental.pallas import tpu_sc as plsc

_NC = 2
_NS = 16
_NW = _NC * _NS


def _sc_gather(table, idx, lo, e):
    n, d = table.shape
    per_w = e // _NW
    ch = 80
    n_ch = per_w // ch
    assert per_w % ch == 0 and per_w * _NW == e

    mesh = plsc.VectorSubcoreMesh(core_axis_name="c", subcore_axis_name="s",
                                  num_cores=_NC, num_subcores=_NS)

    spc = next(s for s in (5, 4, 3, 2, 1) if n_ch % s == 0)
    scr = ch * spc
    nsc = per_w // scr
    assert per_w % scr == 0

    @functools.partial(
        pl.kernel,
        out_type=jax.ShapeDtypeStruct((e, d), jnp.float32),
        mesh=mesh,
        scratch_types=[
            pltpu.VMEM((per_w,), jnp.int32),
            pltpu.VMEM((2, scr, d), jnp.float32),
            pltpu.SemaphoreType.DMA,
            pltpu.SemaphoreType.DMA,
            pltpu.SemaphoreType.DMA,
            pltpu.SemaphoreType.DMA,
        ],
    )
    def k(table_hbm, idx_hbm, out_hbm, idx_all, bufs, g0, g1, w0, w1):
        wid = lax.axis_index("s") * _NC + lax.axis_index("c")
        base = wid * per_w
        gsem = (g0, g1)
        wsem = (w0, w1)

        pltpu.sync_copy(idx_hbm.at[pl.ds(lo + base, per_w)], idx_all)

        wdesc = [None, None]
        for c in range(nsc):
            b = c % 2
            rows = bufs.at[b]
            if wdesc[b] is not None:
                wdesc[b].wait()
            gd = [
                pltpu.async_copy(
                    table_hbm.at[idx_all.at[pl.ds(c * scr + j * ch, ch)]],
                    rows.at[pl.ds(j * ch, ch)], gsem[b])
                for j in range(spc)
            ]
            for dsc in gd:
                dsc.wait()
            wdesc[b] = pltpu.async_copy(
                rows, out_hbm.at[pl.ds(base + c * scr, scr)], wsem[b])
        wdesc[0].wait()
        wdesc[1].wait()

    return k(table, idx)


def _sc_scatter_add(msgs, dst, zeros, n_pad, lo):
    e, d = msgs.shape
    per_w = e // _NW
    ch = 80
    n_ch = per_w // ch
    rows_per_tile = n_pad // _NS
    assert per_w % ch == 0 and rows_per_tile % 8 == 0

    mesh = plsc.VectorSubcoreMesh(core_axis_name="c", subcore_axis_name="s",
                                  num_cores=_NC, num_subcores=_NS)

    @functools.partial(
        pl.kernel,
        out_type=jax.ShapeDtypeStruct((_NC, n_pad, d), jnp.float32),
        mesh=mesh,
        scratch_types=[
            pltpu.VMEM((per_w,), jnp.int32),
            pltpu.VMEM((2, 1, ch), jnp.int32),
            pltpu.VMEM((2, ch, d), jnp.float32),
            pltpu.VMEM_SHARED((n_pad, d), jnp.float32),
            pltpu.SemaphoreType.DMA,
            pltpu.SemaphoreType.DMA,
            pltpu.SemaphoreType.DMA,
            pltpu.SemaphoreType.DMA,
        ],
    )
    def k(msgs_hbm, dst_hbm, zeros_hbm, out_hbm, dst_all, idx_s, bufs, acc_sh,
          l0, l1, s0, s1):
        cid = lax.axis_index("c")
        sid = lax.axis_index("s")
        wid = sid * _NC + cid
        my_rows = pl.ds(sid * rows_per_tile, rows_per_tile)
        lsem = (l0, l1)
        ssem = (s0, s1)
        base = wid * per_w

        pltpu.sync_copy(zeros_hbm, acc_sh.at[my_rows])
        pltpu.sync_copy(dst_hbm.at[pl.ds(lo + base, per_w)], dst_all)
        plsc.subcore_barrier()

        ldesc = [None, None]
        sdesc = [None, None]
        ldesc[0] = pltpu.async_copy(
            msgs_hbm.at[pl.ds(base, ch)], bufs.at[0], lsem[0])
        for c in range(n_ch):
            b = c % 2
            nb = (c + 1) % 2
            for q in range(ch // 16):
                idx_s[b, 0, pl.ds(q * 16, 16)] = (
                    dst_all[pl.ds(c * ch + q * 16, 16)])
            if c + 1 < n_ch:
                if sdesc[nb] is not None:
                    sdesc[nb].wait()
                ldesc[nb] = pltpu.async_copy(
                    msgs_hbm.at[pl.ds(base + (c + 1) * ch, ch)],
                    bufs.at[nb], lsem[nb])
            ldesc[b].wait()
            sdesc[b] = pltpu.async_copy(
                bufs.at[b], acc_sh.at[idx_s.at[b, 0]], ssem[b], add=True)
        sdesc[0].wait()
        sdesc[1].wait()
        plsc.subcore_barrier()
        pltpu.sync_copy(acc_sh.at[my_rows], out_hbm.at[cid, my_rows])

    return k(msgs, dst, zeros)


def _tc_mlp(ty_row, h_src, eaT_aug, w1h_cat, w1a_aug, w2_cat, expand4, b2,
            lo):
    e, h = h_src.shape
    t = b2.shape[0]
    th = w2_cat.shape[0]
    ea1 = eaT_aug.shape[0]
    be = 2560
    assert e % be == 0 and lo % be == 0
    blk0 = lo // be
    grid = (e // be,)

    def body(ty_ref, hs_ref, ea_ref, w1h_ref, w1a_ref, w2_ref, ex4_ref,
             b2_ref, out_ref):
        hs = hs_ref[...].astype(jnp.bfloat16)
        eat = ea_ref[...].astype(jnp.bfloat16)
        ty = ty_ref[...]
        cd0 = (((0,), (0,)), ((), ()))
        x1 = jnp.dot(hs, w1h_ref[...], preferred_element_type=jnp.float32)
        x1 = x1 + lax.dot_general(eat, w1a_ref[...], cd0,
                                  preferred_element_type=jnp.float32)
        x1 = jnp.maximum(x1, 0.0)
        tycol = lax.broadcasted_iota(jnp.int32, (t, 1), 0).astype(jnp.float32)
        m4 = jnp.where(ty == tycol, 1.0, 0.0).astype(jnp.bfloat16)
        mask = lax.dot_general(m4, ex4_ref[...], cd0,
                               preferred_element_type=jnp.float32)
        xm = (x1 * mask).astype(jnp.bfloat16)
        m = jnp.dot(xm, w2_ref[...], preferred_element_type=jnp.float32)
        m = m + lax.dot_general(m4, b2_ref[...].astype(jnp.bfloat16), cd0,
                                preferred_element_type=jnp.float32)
        out_ref[...] = m

    return pl.pallas_call(
        body,
        grid=grid,
        in_specs=[
            pl.BlockSpec((1, be), lambda i: (0, i + blk0)),
            pl.BlockSpec((be, h), lambda i: (i, 0)),
            pl.BlockSpec((ea1, be), lambda i: (0, i + blk0)),
            pl.BlockSpec((h, th), lambda i: (0, 0)),
            pl.BlockSpec((ea1, th), lambda i: (0, 0)),
            pl.BlockSpec((th, h), lambda i: (0, 0)),
            pl.BlockSpec((t, th), lambda i: (0, 0)),
            pl.BlockSpec((t, h), lambda i: (0, 0)),
        ],
        out_specs=pl.BlockSpec((be, h), lambda i: (i, 0)),
        out_shape=jax.ShapeDtypeStruct((e, h), jnp.float32),
    )(ty_row, h_src, eaT_aug, w1h_cat, w1a_aug, w2_cat, expand4, b2)


def _tc_gru(msgs_parts, h, w_ih, w_hh, b_ih, b_hh):
    n, hd = h.shape
    bn = 1000
    grid = (n // bn,)
    np_ = len(msgs_parts)

    def body(*refs):
        part_refs = refs[:np_]
        h_ref, wih_ref, whh_ref, bih_ref, bhh_ref, out_ref = refs[np_:]
        msg = part_refs[0][0] + part_refs[0][1]
        for pr in part_refs[1:]:
            msg = msg + pr[0] + pr[1]
        hv = h_ref[...]
        dims = (((1,), (1,)), ((), ()))
        gi = lax.dot_general(msg.astype(jnp.bfloat16), wih_ref[...], dims,
                             preferred_element_type=jnp.float32)
        gi = gi + bih_ref[...]
        gh = lax.dot_general(hv.astype(jnp.bfloat16), whh_ref[...], dims,
                             preferred_element_type=jnp.float32)
        gh = gh + bhh_ref[...]
        r = jax.nn.sigmoid(gi[:, :hd] + gh[:, :hd])
        z = jax.nn.sigmoid(gi[:, hd:2 * hd] + gh[:, hd:2 * hd])
        nn = jnp.tanh(gi[:, 2 * hd:] + r * gh[:, 2 * hd:])
        out_ref[...] = (1.0 - z) * nn + z * hv

    return pl.pallas_call(
        body,
        grid=grid,
        in_specs=(
            [pl.BlockSpec((2, bn, hd), lambda i: (0, i, 0))
             for _ in range(np_)]
            + [
                pl.BlockSpec((bn, hd), lambda i: (i, 0)),
                pl.BlockSpec((3 * hd, hd), lambda i: (0, 0)),
                pl.BlockSpec((3 * hd, hd), lambda i: (0, 0)),
                pl.BlockSpec((1, 3 * hd), lambda i: (0, 0)),
                pl.BlockSpec((1, 3 * hd), lambda i: (0, 0)),
            ]
        ),
        out_specs=pl.BlockSpec((bn, hd), lambda i: (i, 0)),
        out_shape=jax.ShapeDtypeStruct((n, hd), jnp.float32),
    )(*msgs_parts, h, w_ih, w_hh, b_ih, b_hh)


def kernel(h, edge_index, edge_type, edge_attr, W1, b1, W2, b2,
           gru_w_ih, gru_w_hh, gru_b_ih, gru_b_hh):
    n, hd = h.shape
    e = edge_index.shape[1]

    src = edge_index[0].astype(jnp.int32)
    dst = edge_index[1].astype(jnp.int32)
    ty_row = edge_type.astype(jnp.float32).reshape(1, e)

    t = W1.shape[0]
    th = t * hd
    w1h_cat = W1[:, :hd, :].transpose(1, 0, 2).reshape(hd, th)
    w1a_cat = W1[:, hd:, :].transpose(1, 0, 2).reshape(-1, th)
    b1_cat = b1.reshape(1, th)
    w1a_aug = jnp.concatenate([w1a_cat, b1_cat], axis=0)
    eaT_aug = jnp.concatenate(
        [edge_attr.T, jnp.ones((1, e), jnp.float32)], axis=0)
    w2_cat = W2.reshape(th, hd)
    expand4 = jnp.repeat(jnp.eye(t, dtype=jnp.bfloat16), hd, axis=1)
    w1h_bf = w1h_cat.astype(jnp.bfloat16)
    w1a_bf = w1a_aug.astype(jnp.bfloat16)
    w2_bf = w2_cat.astype(jnp.bfloat16)

    n_pad = ((n + 8 * _NS - 1) // (8 * _NS)) * (8 * _NS)
    zeros = jnp.zeros((n_pad // _NS, hd), jnp.float32)

    nblk = e // 2560
    bcuts = [0, nblk * 21 // 100, nblk // 2, nblk - nblk * 21 // 100, nblk]
    cuts = [b * 2560 for b in bcuts]
    partials = []
    msgs_chunks = []
    for lo, hi in zip(cuts[:-1], cuts[1:]):
        h_src = _sc_gather(h, src, lo, hi - lo)
        msgs = _tc_mlp(ty_row, h_src, eaT_aug,
                       w1h_bf, w1a_bf, w2_bf, expand4, b2, lo)
        msgs_chunks.append((msgs, lo))
    for msgs, lo in msgs_chunks:
        partials.append(_sc_scatter_add(msgs, dst, zeros, n_pad, lo))

    return _tc_gru(partials, h,
                   gru_w_ih.astype(jnp.bfloat16), gru_w_hh.astype(jnp.bfloat16),
                   gru_b_ih.reshape(1, 3 * hd), gru_b_hh.reshape(1, 3 * hd))


# speedup vs baseline: 7.4507x; 1.0312x over previous
import functools

import jax
import jax.numpy as jnp
from jax import lax
from jax.experimental import pallas as pl
from jax.experimental.pallas import tpu as pltpu
from jax.experimental.pallas import tpu_sc as plsc

_NC = 2
_NS = 16
_NW = _NC * _NS


def _sc_gather(table, idx, lo, e):
    n, d = table.shape
    per_w = e // _NW
    ch = 80
    n_ch = per_w // ch
    assert per_w % ch == 0 and per_w * _NW == e

    mesh = plsc.VectorSubcoreMesh(core_axis_name="c", subcore_axis_name="s",
                                  num_cores=_NC, num_subcores=_NS)

    spc = next(s for s in (5, 4, 3, 2, 1) if n_ch % s == 0)
    scr = ch * spc
    nsc = per_w // scr
    assert per_w % scr == 0

    @functools.partial(
        pl.kernel,
        out_type=jax.ShapeDtypeStruct((e, d), jnp.float32),
        mesh=mesh,
        scratch_types=[
            pltpu.VMEM((per_w,), jnp.int32),
            pltpu.VMEM((2, scr, d), jnp.float32),
            pltpu.SemaphoreType.DMA,
            pltpu.SemaphoreType.DMA,
            pltpu.SemaphoreType.DMA,
            pltpu.SemaphoreType.DMA,
        ],
    )
    def k(table_hbm, idx_hbm, out_hbm, idx_all, bufs, g0, g1, w0, w1):
        wid = lax.axis_index("s") * _NC + lax.axis_index("c")
        base = wid * per_w
        gsem = (g0, g1)
        wsem = (w0, w1)

        pltpu.sync_copy(idx_hbm.at[pl.ds(lo + base, per_w)], idx_all)

        wdesc = [None, None]
        for c in range(nsc):
            b = c % 2
            rows = bufs.at[b]
            if wdesc[b] is not None:
                wdesc[b].wait()
            gd = [
                pltpu.async_copy(
                    table_hbm.at[idx_all.at[pl.ds(c * scr + j * ch, ch)]],
                    rows.at[pl.ds(j * ch, ch)], gsem[b])
                for j in range(spc)
            ]
            for dsc in gd:
                dsc.wait()
            wdesc[b] = pltpu.async_copy(
                rows, out_hbm.at[pl.ds(base + c * scr, scr)], wsem[b])
        wdesc[0].wait()
        wdesc[1].wait()

    return k(table, idx)


def _sc_scatter_add(msgs, dst, zeros, n_pad, lo):
    e, d = msgs.shape
    per_w = e // _NW
    ch = 80
    n_ch = per_w // ch
    rows_per_tile = n_pad // _NS
    assert per_w % ch == 0 and rows_per_tile % 8 == 0

    mesh = plsc.VectorSubcoreMesh(core_axis_name="c", subcore_axis_name="s",
                                  num_cores=_NC, num_subcores=_NS)

    @functools.partial(
        pl.kernel,
        out_type=jax.ShapeDtypeStruct((_NC, n_pad, d), jnp.float32),
        mesh=mesh,
        scratch_types=[
            pltpu.VMEM((per_w,), jnp.int32),
            pltpu.VMEM((2, 1, ch), jnp.int32),
            pltpu.VMEM((2, ch, d), jnp.float32),
            pltpu.VMEM_SHARED((n_pad, d), jnp.float32),
            pltpu.SemaphoreType.DMA,
            pltpu.SemaphoreType.DMA,
            pltpu.SemaphoreType.DMA,
            pltpu.SemaphoreType.DMA,
        ],
    )
    def k(msgs_hbm, dst_hbm, zeros_hbm, out_hbm, dst_all, idx_s, bufs, acc_sh,
          l0, l1, s0, s1):
        cid = lax.axis_index("c")
        sid = lax.axis_index("s")
        wid = sid * _NC + cid
        my_rows = pl.ds(sid * rows_per_tile, rows_per_tile)
        lsem = (l0, l1)
        ssem = (s0, s1)
        base = wid * per_w

        pltpu.sync_copy(zeros_hbm, acc_sh.at[my_rows])
        pltpu.sync_copy(dst_hbm.at[pl.ds(lo + base, per_w)], dst_all)
        plsc.subcore_barrier()

        ldesc = [None, None]
        sdesc = [None, None]
        ldesc[0] = pltpu.async_copy(
            msgs_hbm.at[pl.ds(base, ch)], bufs.at[0], lsem[0])
        for c in range(n_ch):
            b = c % 2
            nb = (c + 1) % 2
            for q in range(ch // 16):
                idx_s[b, 0, pl.ds(q * 16, 16)] = (
                    dst_all[pl.ds(c * ch + q * 16, 16)])
            if c + 1 < n_ch:
                if sdesc[nb] is not None:
                    sdesc[nb].wait()
                ldesc[nb] = pltpu.async_copy(
                    msgs_hbm.at[pl.ds(base + (c + 1) * ch, ch)],
                    bufs.at[nb], lsem[nb])
            ldesc[b].wait()
            sdesc[b] = pltpu.async_copy(
                bufs.at[b], acc_sh.at[idx_s.at[b, 0]], ssem[b], add=True)
        sdesc[0].wait()
        sdesc[1].wait()
        plsc.subcore_barrier()
        pltpu.sync_copy(acc_sh.at[my_rows], out_hbm.at[cid, my_rows])

    return k(msgs, dst, zeros)


def _tc_mlp(ty_row, h_src, eaT_aug, w1h_cat, w1a_aug, w2_cat, expand4, b2,
            lo):
    e, h = h_src.shape
    t = b2.shape[0]
    th = w2_cat.shape[0]
    ea1 = eaT_aug.shape[0]
    be = 2560
    assert e % be == 0 and lo % be == 0
    blk0 = lo // be
    grid = (e // be,)

    def body(ty_ref, hs_ref, ea_ref, w1h_ref, w1a_ref, w2_ref, ex4_ref,
             b2_ref, out_ref):
        hs = hs_ref[...].astype(jnp.bfloat16)
        eat = ea_ref[...].astype(jnp.bfloat16)
        ty = ty_ref[...]
        cd0 = (((0,), (0,)), ((), ()))
        x1 = jnp.dot(hs, w1h_ref[...], preferred_element_type=jnp.float32)
        x1 = x1 + lax.dot_general(eat, w1a_ref[...], cd0,
                                  preferred_element_type=jnp.float32)
        x1 = jnp.maximum(x1, 0.0)
        tycol = lax.broadcasted_iota(jnp.int32, (t, 1), 0).astype(jnp.float32)
        m4 = jnp.where(ty == tycol, 1.0, 0.0).astype(jnp.bfloat16)
        mask = lax.dot_general(m4, ex4_ref[...], cd0,
                               preferred_element_type=jnp.float32)
        xm = (x1 * mask).astype(jnp.bfloat16)
        m = jnp.dot(xm, w2_ref[...], preferred_element_type=jnp.float32)
        m = m + lax.dot_general(m4, b2_ref[...].astype(jnp.bfloat16), cd0,
                                preferred_element_type=jnp.float32)
        out_ref[...] = m

    return pl.pallas_call(
        body,
        grid=grid,
        in_specs=[
            pl.BlockSpec((1, be), lambda i: (0, i + blk0)),
            pl.BlockSpec((be, h), lambda i: (i, 0)),
            pl.BlockSpec((ea1, be), lambda i: (0, i + blk0)),
            pl.BlockSpec((h, th), lambda i: (0, 0)),
            pl.BlockSpec((ea1, th), lambda i: (0, 0)),
            pl.BlockSpec((th, h), lambda i: (0, 0)),
            pl.BlockSpec((t, th), lambda i: (0, 0)),
            pl.BlockSpec((t, h), lambda i: (0, 0)),
        ],
        out_specs=pl.BlockSpec((be, h), lambda i: (i, 0)),
        out_shape=jax.ShapeDtypeStruct((e, h), jnp.float32),
    )(ty_row, h_src, eaT_aug, w1h_cat, w1a_aug, w2_cat, expand4, b2)


def _tc_gru(msgs_parts, h, w_ih, w_hh, b_ih, b_hh):
    n, hd = h.shape
    bn = 1000
    grid = (n // bn,)
    np_ = len(msgs_parts)

    def body(*refs):
        part_refs = refs[:np_]
        h_ref, wih_ref, whh_ref, bih_ref, bhh_ref, out_ref = refs[np_:]
        msg = part_refs[0][0] + part_refs[0][1]
        for pr in part_refs[1:]:
            msg = msg + pr[0] + pr[1]
        hv = h_ref[...]
        dims = (((1,), (1,)), ((), ()))
        gi = lax.dot_general(msg.astype(jnp.bfloat16), wih_ref[...], dims,
                             preferred_element_type=jnp.float32)
        gi = gi + bih_ref[...]
        gh = lax.dot_general(hv.astype(jnp.bfloat16), whh_ref[...], dims,
                             preferred_element_type=jnp.float32)
        gh = gh + bhh_ref[...]
        r = jax.nn.sigmoid(gi[:, :hd] + gh[:, :hd])
        z = jax.nn.sigmoid(gi[:, hd:2 * hd] + gh[:, hd:2 * hd])
        nn = jnp.tanh(gi[:, 2 * hd:] + r * gh[:, 2 * hd:])
        out_ref[...] = (1.0 - z) * nn + z * hv

    return pl.pallas_call(
        body,
        grid=grid,
        in_specs=(
            [pl.BlockSpec((2, bn, hd), lambda i: (0, i, 0))
             for _ in range(np_)]
            + [
                pl.BlockSpec((bn, hd), lambda i: (i, 0)),
                pl.BlockSpec((3 * hd, hd), lambda i: (0, 0)),
                pl.BlockSpec((3 * hd, hd), lambda i: (0, 0)),
                pl.BlockSpec((1, 3 * hd), lambda i: (0, 0)),
                pl.BlockSpec((1, 3 * hd), lambda i: (0, 0)),
            ]
        ),
        out_specs=pl.BlockSpec((bn, hd), lambda i: (i, 0)),
        out_shape=jax.ShapeDtypeStruct((n, hd), jnp.float32),
    )(*msgs_parts, h, w_ih, w_hh, b_ih, b_hh)


def kernel(h, edge_index, edge_type, edge_attr, W1, b1, W2, b2,
           gru_w_ih, gru_w_hh, gru_b_ih, gru_b_hh):
    n, hd = h.shape
    e = edge_index.shape[1]

    src = edge_index[0].astype(jnp.int32)
    dst = edge_index[1].astype(jnp.int32)
    ty_row = edge_type.astype(jnp.float32).reshape(1, e)

    t = W1.shape[0]
    th = t * hd
    w1h_cat = W1[:, :hd, :].transpose(1, 0, 2).reshape(hd, th)
    w1a_cat = W1[:, hd:, :].transpose(1, 0, 2).reshape(-1, th)
    b1_cat = b1.reshape(1, th)
    w1a_aug = jnp.concatenate([w1a_cat, b1_cat], axis=0)
    eaT_aug = jnp.concatenate(
        [edge_attr.T, jnp.ones((1, e), jnp.float32)], axis=0)
    w2_cat = W2.reshape(th, hd)
    expand4 = jnp.repeat(jnp.eye(t, dtype=jnp.bfloat16), hd, axis=1)
    w1h_bf = w1h_cat.astype(jnp.bfloat16)
    w1a_bf = w1a_aug.astype(jnp.bfloat16)
    w2_bf = w2_cat.astype(jnp.bfloat16)

    n_pad = ((n + 8 * _NS - 1) // (8 * _NS)) * (8 * _NS)
    zeros = jnp.zeros((n_pad // _NS, hd), jnp.float32)

    nblk = e // 2560
    b1_ = nblk * 14 // 100
    bcuts = [0, b1_, nblk * 40 // 100, nblk * 66 // 100, nblk - b1_, nblk]
    cuts = [b * 2560 for b in bcuts]
    partials = []
    msgs_chunks = []
    for lo, hi in zip(cuts[:-1], cuts[1:]):
        h_src = _sc_gather(h, src, lo, hi - lo)
        msgs = _tc_mlp(ty_row, h_src, eaT_aug,
                       w1h_bf, w1a_bf, w2_bf, expand4, b2, lo)
        msgs_chunks.append((msgs, lo))
    for msgs, lo in msgs_chunks:
        partials.append(_sc_scatter_add(msgs, dst, zeros, n_pad, lo))

    return _tc_gru(partials, h,
                   gru_w_ih.astype(jnp.bfloat16), gru_w_hh.astype(jnp.bfloat16),
                   gru_b_ih.reshape(1, 3 * hd), gru_b_hh.reshape(1, 3 * hd))
